# baseline (speedup 1.0000x reference)
import jax
import jax.numpy as jnp
from jax import lax
from jax.experimental import pallas as pl
from jax.experimental.pallas import tpu as pltpu, tpu_sc as plsc

N_NODES = 100000
IN_CH = 25
OUT_CH = 50
HALF = 16

NC = 2
NS = 16
B = 128
J = 6
G = 261
CHUNK = J * B
E_TILE = CHUNK * G
E_PAD = NS * E_TILE
ROWS_PER_TILE = 6272
N_ACC = NS * ROWS_PER_TILE


ROWS_E = E_TILE // B


def _sc_body(src_hbm, dst_hbm, tab0_hbm, tab1_hbm, out_hbm, eidx_v, rows, acc,
             sem_i, sem_g, sem_s):
    c = lax.axis_index("c")
    s = lax.axis_index("s")

    def zero_row(i, carry):
        rows[i, :] = jnp.zeros((16,), jnp.float32)
        return carry

    lax.fori_loop(0, 2 * CHUNK, zero_row, 0)
    base = s * ROWS_PER_TILE
    for k in range(ROWS_PER_TILE // (2 * CHUNK)):
        pltpu.sync_copy(rows, acc.at[pl.ds(base + k * 2 * CHUNK, 2 * CHUNK)])
    tail = ROWS_PER_TILE % (2 * CHUNK)
    if tail:
        pltpu.sync_copy(rows.at[pl.ds(0, tail)],
                        acc.at[pl.ds(base + ROWS_PER_TILE - tail, tail)])
    plsc.subcore_barrier()

    ebase = s * ROWS_E

    def _drain(semx):
        pltpu.make_async_copy(tab0_hbm.at[pl.ds(0, CHUNK)],
                              rows.at[pl.ds(0, CHUNK)], semx).wait()

    def _drain_idx():
        pltpu.make_async_copy(src_hbm.at[pl.ds(0, J)], eidx_v.at[0, 0],
                              sem_i).wait()
        pltpu.make_async_copy(src_hbm.at[pl.ds(0, J)], eidx_v.at[0, 1],
                              sem_i).wait()

    def _load_idx(g, slot, copy):
        copy(src_hbm.at[pl.ds(ebase + g * J, J)], eidx_v.at[slot, 0])
        copy(dst_hbm.at[pl.ds(ebase + g * J, J)], eidx_v.at[slot, 1])

    def _fire_gathers(slot, roff):
        @pl.when(c == 0)
        def _g0():
            for j in range(J):
                pltpu.async_copy(tab0_hbm.at[eidx_v.at[slot, 0, j]],
                                 rows.at[pl.ds(roff + j * B, B)], sem_g)

        @pl.when(c == 1)
        def _g1():
            for j in range(J):
                pltpu.async_copy(tab1_hbm.at[eidx_v.at[slot, 0, j]],
                                 rows.at[pl.ds(roff + j * B, B)], sem_g)

    _load_idx(0, 0, pltpu.sync_copy)
    _load_idx(1, 1, lambda a, b: pltpu.async_copy(a, b, sem_i))
    _fire_gathers(0, 0)

    def step(g, carry):
        p = g & 1
        roff = p * CHUNK
        i1 = (g + 1) % 3
        i2 = (g + 2) % 3
        ig = g % 3
        _drain_idx()

        @pl.when(g > 0)
        def _w1():
            _drain(sem_s)

        _load_idx(g + 2, i2, lambda a, b: pltpu.async_copy(a, b, sem_i))
        _drain(sem_g)

        @pl.when(g < G - 1)
        def _w2():
            _fire_gathers(i1, CHUNK - roff)

        for j in range(J):
            pltpu.async_copy(rows.at[pl.ds(roff + j * B, B)],
                             acc.at[eidx_v.at[ig, 1, j]], sem_s, add=True)
        return carry

    lax.fori_loop(0, G, step, 0)
    _drain(sem_s)
    _drain_idx()
    plsc.subcore_barrier()

    for k in range(ROWS_PER_TILE // CHUNK):
        pltpu.sync_copy(acc.at[pl.ds(base + k * CHUNK, CHUNK)],
                        out_hbm.at[c, pl.ds(base + k * CHUNK, CHUNK)])
    tail2 = ROWS_PER_TILE % CHUNK
    if tail2:
        pltpu.sync_copy(acc.at[pl.ds(base + ROWS_PER_TILE - tail2, tail2)],
                        out_hbm.at[c, pl.ds(base + ROWS_PER_TILE - tail2,
                                            tail2)])


_sc_agg = pl.kernel(
    _sc_body,
    out_type=jax.ShapeDtypeStruct((NC, N_ACC, HALF), jnp.float32),
    mesh=plsc.VectorSubcoreMesh(core_axis_name="c", subcore_axis_name="s"),
    scratch_types=[
        pltpu.VMEM((3, 2, J, B), jnp.int32),
        pltpu.VMEM((2 * CHUNK, HALF), jnp.float32),
        pltpu.VMEM_SHARED((N_ACC, HALF), jnp.float32),
        pltpu.SemaphoreType.DMA,
        pltpu.SemaphoreType.DMA,
        pltpu.SemaphoreType.DMA,
    ],
    compiler_params=pltpu.CompilerParams(use_tc_tiling_on_sc=False),
)


def _mm_body(a0_ref, a1_ref, w0_ref, w1_ref, b_ref, o_ref):
    acc = jnp.dot(a0_ref[...], w0_ref[...], preferred_element_type=jnp.float32)
    acc = acc + jnp.dot(a1_ref[...], w1_ref[...],
                        preferred_element_type=jnp.float32)
    o_ref[...] = jnp.maximum(acc + b_ref[...], 0.0)


_BMW = 128

_mm = pl.pallas_call(
    _mm_body,
    grid=(N_ACC // 8 // _BMW,),
    in_specs=[
        pl.BlockSpec((_BMW, 8 * HALF), lambda i: (i, 0)),
        pl.BlockSpec((_BMW, 8 * HALF), lambda i: (i, 0)),
        pl.BlockSpec((8 * HALF, 8 * OUT_CH), lambda i: (0, 0)),
        pl.BlockSpec((8 * HALF, 8 * OUT_CH), lambda i: (0, 0)),
        pl.BlockSpec((1, 8 * OUT_CH), lambda i: (0, 0)),
    ],
    out_specs=pl.BlockSpec((_BMW, 8 * OUT_CH), lambda i: (i, 0)),
    out_shape=jax.ShapeDtypeStruct((N_ACC // 8, 8 * OUT_CH), jnp.float32),
)


def kernel(x, edge_index, W, b):
    x = x.astype(jnp.float32)
    src = edge_index[0].astype(jnp.int32)
    dst = edge_index[1].astype(jnp.int32)

    tab0 = x[:, :HALF]
    tab1 = jnp.pad(x[:, HALF:], ((0, 0), (0, 2 * HALF - IN_CH)))

    pad = E_PAD + 2 * CHUNK - src.shape[0]
    src_p = jnp.concatenate([src, jnp.zeros((pad,), jnp.int32)])
    garbage = N_NODES + (jnp.arange(pad, dtype=jnp.int32) % (N_ACC - N_NODES))
    dst_p = jnp.concatenate([dst, garbage])
    src2d = src_p.reshape(-1, B)
    dst2d = dst_p.reshape(-1, B)

    agg = _sc_agg(src2d, dst2d, tab0, tab1)
    agg = agg.reshape(NC, N_ACC // 8, 8 * HALF)

    eye8 = jnp.eye(8, dtype=jnp.float32)
    w0 = jnp.kron(eye8, W[:HALF].astype(jnp.float32))
    w1 = jnp.kron(eye8, jnp.pad(W[HALF:].astype(jnp.float32),
                                ((0, 2 * HALF - IN_CH), (0, 0))))
    bw = jnp.tile(b.astype(jnp.float32), 8).reshape(1, 8 * OUT_CH)
    out = _mm(agg[0], agg[1], w0, w1, bw)
    return out.reshape(N_ACC, OUT_CH)[:N_NODES]


# speedup vs baseline: 1.0090x; 1.0090x over previous
import jax
import jax.numpy as jnp
from jax import lax
from jax.experimental import pallas as pl
from jax.experimental.pallas import tpu as pltpu, tpu_sc as plsc

N_NODES = 100000
IN_CH = 25
OUT_CH = 50
HALF = 16

NC = 2
NS = 16
B = 128
J = 6
G = 261
CHUNK = J * B
E_TILE = CHUNK * G
E_PAD = NS * E_TILE
ROWS_PER_TILE = 6272
N_ACC = NS * ROWS_PER_TILE


ROWS_E = E_TILE // B


def _sc_body(src_hbm, dst_hbm, tab0_hbm, tab1_hbm, out_hbm, eidx_v, rows, acc,
             sem_i, sem_g, sem_s):
    c = lax.axis_index("c")
    s = lax.axis_index("s")

    def zero_row(i, carry):
        rows[i, :] = jnp.zeros((16,), jnp.float32)
        return carry

    lax.fori_loop(0, 2 * CHUNK, zero_row, 0)
    base = s * ROWS_PER_TILE
    for k in range(ROWS_PER_TILE // (2 * CHUNK)):
        pltpu.sync_copy(rows, acc.at[pl.ds(base + k * 2 * CHUNK, 2 * CHUNK)])
    tail = ROWS_PER_TILE % (2 * CHUNK)
    if tail:
        pltpu.sync_copy(rows.at[pl.ds(0, tail)],
                        acc.at[pl.ds(base + ROWS_PER_TILE - tail, tail)])
    plsc.subcore_barrier()

    ebase = s * ROWS_E

    def _drain(semx):
        pltpu.make_async_copy(tab0_hbm.at[pl.ds(0, CHUNK)],
                              rows.at[pl.ds(0, CHUNK)], semx).wait()

    def _drain_idx():
        pltpu.make_async_copy(src_hbm.at[pl.ds(0, J)], eidx_v.at[0, 0],
                              sem_i).wait()
        pltpu.make_async_copy(src_hbm.at[pl.ds(0, J)], eidx_v.at[0, 1],
                              sem_i).wait()

    def _load_idx(g, slot, copy):
        copy(src_hbm.at[pl.ds(ebase + g * J, J)], eidx_v.at[slot, 0])
        copy(dst_hbm.at[pl.ds(ebase + g * J, J)], eidx_v.at[slot, 1])

    def _fire_gathers(slot, roff):
        @pl.when(c == 0)
        def _g0():
            for j in range(J):
                pltpu.async_copy(tab0_hbm.at[eidx_v.at[slot, 0, j]],
                                 rows.at[pl.ds(roff + j * B, B)], sem_g)

        @pl.when(c == 1)
        def _g1():
            for j in range(J):
                pltpu.async_copy(tab1_hbm.at[eidx_v.at[slot, 0, j]],
                                 rows.at[pl.ds(roff + j * B, B)], sem_g)

    _load_idx(0, 0, pltpu.sync_copy)
    _load_idx(1, 1, lambda a, b: pltpu.async_copy(a, b, sem_i))
    _fire_gathers(0, 0)

    def step(g, carry):
        p = g & 1
        roff = p * CHUNK
        i1 = (g + 1) % 3
        i2 = (g + 2) % 3
        ig = g % 3
        _drain_idx()

        @pl.when(g > 0)
        def _w1():
            _drain(sem_s)

        _load_idx(g + 2, i2, lambda a, b: pltpu.async_copy(a, b, sem_i))
        _drain(sem_g)

        @pl.when(g < G - 1)
        def _w2():
            _fire_gathers(i1, CHUNK - roff)

        for j in range(J):
            pltpu.async_copy(rows.at[pl.ds(roff + j * B, B)],
                             acc.at[eidx_v.at[ig, 1, j]], sem_s, add=True)
        return carry

    lax.fori_loop(0, G, step, 0)
    _drain(sem_s)
    _drain_idx()
    plsc.subcore_barrier()

    for k in range(ROWS_PER_TILE // CHUNK):
        pltpu.sync_copy(acc.at[pl.ds(base + k * CHUNK, CHUNK)],
                        out_hbm.at[c, pl.ds(base + k * CHUNK, CHUNK)])
    tail2 = ROWS_PER_TILE % CHUNK
    if tail2:
        pltpu.sync_copy(acc.at[pl.ds(base + ROWS_PER_TILE - tail2, tail2)],
                        out_hbm.at[c, pl.ds(base + ROWS_PER_TILE - tail2,
                                            tail2)])


_sc_agg = pl.kernel(
    _sc_body,
    out_type=jax.ShapeDtypeStruct((NC, N_ACC, HALF), jnp.float32),
    mesh=plsc.VectorSubcoreMesh(core_axis_name="c", subcore_axis_name="s"),
    scratch_types=[
        pltpu.VMEM((3, 2, J, B), jnp.int32),
        pltpu.VMEM((2 * CHUNK, HALF), jnp.float32),
        pltpu.VMEM_SHARED((N_ACC, HALF), jnp.float32),
        pltpu.SemaphoreType.DMA,
        pltpu.SemaphoreType.DMA,
        pltpu.SemaphoreType.DMA,
    ],
    compiler_params=pltpu.CompilerParams(use_tc_tiling_on_sc=False),
)


def _mm_body(a0_ref, a1_ref, w0_ref, w1_ref, b_ref, o_ref):
    acc = jnp.dot(a0_ref[...], w0_ref[...], preferred_element_type=jnp.float32)
    acc = acc + jnp.dot(a1_ref[...], w1_ref[...],
                        preferred_element_type=jnp.float32)
    o_ref[...] = jnp.maximum(acc + b_ref[...], 0.0)


_BM = 1024

_mm = pl.pallas_call(
    _mm_body,
    grid=(N_ACC // _BM,),
    in_specs=[
        pl.BlockSpec((_BM, HALF), lambda i: (i, 0)),
        pl.BlockSpec((_BM, HALF), lambda i: (i, 0)),
        pl.BlockSpec((HALF, OUT_CH), lambda i: (0, 0)),
        pl.BlockSpec((HALF, OUT_CH), lambda i: (0, 0)),
        pl.BlockSpec((1, OUT_CH), lambda i: (0, 0)),
    ],
    out_specs=pl.BlockSpec((_BM, OUT_CH), lambda i: (i, 0)),
    out_shape=jax.ShapeDtypeStruct((N_ACC, OUT_CH), jnp.float32),
)


def kernel(x, edge_index, W, b):
    x = x.astype(jnp.float32)
    src = edge_index[0].astype(jnp.int32)
    dst = edge_index[1].astype(jnp.int32)

    tab0 = x[:, :HALF]
    tab1 = jnp.pad(x[:, HALF:], ((0, 0), (0, 2 * HALF - IN_CH)))

    pad = E_PAD + 2 * CHUNK - src.shape[0]
    src_p = jnp.concatenate([src, jnp.zeros((pad,), jnp.int32)])
    garbage = N_NODES + (jnp.arange(pad, dtype=jnp.int32) % (N_ACC - N_NODES))
    dst_p = jnp.concatenate([dst, garbage])
    src2d = src_p.reshape(-1, B)
    dst2d = dst_p.reshape(-1, B)

    agg = _sc_agg(src2d, dst2d, tab0, tab1)

    w0 = W[:HALF].astype(jnp.float32)
    w1 = jnp.pad(W[HALF:].astype(jnp.float32),
                 ((0, 2 * HALF - IN_CH), (0, 0)))
    out = _mm(agg[0], agg[1], w0, w1, b.reshape(1, OUT_CH).astype(jnp.float32))
    return out[:N_NODES]
